# baseline (speedup 1.0000x reference)
import functools
import jax
import jax.numpy as jnp
from jax import lax
from jax.experimental import pallas as pl
from jax.experimental.pallas import tpu as pltpu
from jax.experimental.pallas import tpu_sc as plsc

VOCAB = 1000000
EMBED_DIM = 64
BATCH = 16384

_info = plsc.get_sparse_core_info()
_NC, _NS = _info.num_cores, _info.num_subcores
_NW = _NC * _NS
_BPW = BATCH // _NW

_mesh = plsc.VectorSubcoreMesh(core_axis_name="c", subcore_axis_name="s")


@functools.partial(
    pl.kernel,
    mesh=_mesh,
    out_type=jax.ShapeDtypeStruct((BATCH, EMBED_DIM), jnp.float32),
    scratch_types=[
        pltpu.VMEM((_BPW,), jnp.int32),
        pltpu.VMEM((_BPW, EMBED_DIM), jnp.float32),
        pltpu.SemaphoreType.DMA,
    ],
    compiler_params=pltpu.CompilerParams(needs_layout_passes=False),
)
def _gather(idx_hbm, table_hbm, out_hbm, idx_v, buf_v, sem):
    wid = lax.axis_index("s") * _NC + lax.axis_index("c")
    base = wid * _BPW
    pltpu.sync_copy(idx_hbm.at[pl.ds(base, _BPW)], idx_v)

    def do_chunk(c, _):
        v = idx_v[pl.ds(c * 16, 16)]
        copies = []
        for j in range(16):
            copies.append(
                pltpu.async_copy(
                    table_hbm.at[v[j]], buf_v.at[c * 16 + j], sem
                )
            )
        for cp in copies:
            cp.wait()
        return 0

    lax.fori_loop(0, _BPW // 16, do_chunk, 0)
    pltpu.sync_copy(buf_v, out_hbm.at[pl.ds(base, _BPW)])


def kernel(x, table):
    out = _gather(x.reshape(BATCH), table)
    return out.reshape(BATCH, 1, EMBED_DIM)


# speedup vs baseline: 1.5472x; 1.5472x over previous
import functools
import jax
import jax.numpy as jnp
from jax import lax
from jax.experimental import pallas as pl
from jax.experimental.pallas import tpu as pltpu
from jax.experimental.pallas import tpu_sc as plsc

VOCAB = 1000000
EMBED_DIM = 64
BATCH = 16384

_info = plsc.get_sparse_core_info()
_NC, _NS = _info.num_cores, _info.num_subcores
_NW = _NC * _NS
_W = 512
_NBLK = (VOCAB + _W - 1) // _W
_BASE_BLKS = _NBLK // _NW
_EXTRA = _NBLK - _BASE_BLKS * _NW
_RING = 16

_mesh = plsc.VectorSubcoreMesh(core_axis_name="c", subcore_axis_name="s")


@functools.partial(
    pl.kernel,
    mesh=_mesh,
    out_type=jax.ShapeDtypeStruct((BATCH, EMBED_DIM), jnp.float32),
    scratch_types=[
        pltpu.VMEM((BATCH,), jnp.int32),
        pltpu.VMEM((BATCH + 16,), jnp.int32),
        pltpu.VMEM((BATCH + 16,), jnp.int32),
        pltpu.VMEM((EMBED_DIM, _W), jnp.float32),
        pltpu.VMEM((16,), jnp.int32),
        pltpu.VMEM((16,), jnp.int32),
        pltpu.VMEM((_RING, EMBED_DIM), jnp.float32),
        pltpu.SemaphoreType.DMA,
        pltpu.SemaphoreType.DMA,
    ],
    compiler_params=pltpu.CompilerParams(needs_layout_passes=False),
)
def _gather(idx_hbm, tq_hbm, out_hbm, idx_all_v, lval_v, lpos_v, buf_v,
            mval_v, mpos_v, oring_v, sem_in, sem_out):
    wid = lax.axis_index("s") * _NC + lax.axis_index("c")
    blk0 = jnp.where(
        wid < _EXTRA,
        (_BASE_BLKS + 1) * wid,
        _EXTRA * (_BASE_BLKS + 1) + _BASE_BLKS * (wid - _EXTRA),
    )
    nblk = jnp.where(wid < _EXTRA, _BASE_BLKS + 1, _BASE_BLKS)
    lo = blk0 * _W
    hi = (blk0 + nblk) * _W

    pltpu.sync_copy(idx_hbm, idx_all_v)

    def scan_all(g, n):
        v = idx_all_v[pl.ds(g * 16, 16)]
        p = lax.iota(jnp.int32, 16) + g * 16
        m = jnp.logical_and(v >= lo, v < hi)
        plsc.store_compressed(lval_v.at[pl.ds(n, 16)], v, mask=m)
        plsc.store_compressed(lpos_v.at[pl.ds(n, 16)], p, mask=m)
        return n + plsc.all_reduce_population_count(m)[0]

    n = lax.fori_loop(0, BATCH // 16, scan_all, jnp.int32(0))
    ng = (n + 15) // 16

    def do_block(bi, issued):
        blk = blk0 + bi
        r0 = blk * _W
        partial = blk == _NBLK - 1

        @pl.when(jnp.logical_not(partial))
        def _():
            cps = [
                pltpu.async_copy(
                    tq_hbm.at[pl.ds(tc * 8, 8), pl.ds(r0, _W)],
                    buf_v.at[pl.ds(tc * 8, 8)],
                    sem_in,
                )
                for tc in range(EMBED_DIM // 8)
            ]
            for cp in cps:
                cp.wait()

        @pl.when(partial)
        def _():
            tail = VOCAB - (_NBLK - 1) * _W
            cps = [
                pltpu.async_copy(
                    tq_hbm.at[pl.ds(tc * 8, 8), pl.ds(r0, tail)],
                    buf_v.at[pl.ds(tc * 8, 8), pl.ds(0, tail)],
                    sem_in,
                )
                for tc in range(EMBED_DIM // 8)
            ]
            for cp in cps:
                cp.wait()

        def scan_list(g, issued):
            v = lval_v[pl.ds(g * 16, 16)]
            p = lpos_v[pl.ds(g * 16, 16)]
            m = jnp.logical_and(v >= r0, v < r0 + _W)
            plsc.store_compressed(mval_v.at[pl.ds(0, 16)], v, mask=m)
            plsc.store_compressed(mpos_v.at[pl.ds(0, 16)], p, mask=m)
            cnt = plsc.all_reduce_population_count(m)[0]

            def emit(j, issued):
                jv = jnp.full((16,), 0, jnp.int32) + j
                val = plsc.load_gather(mval_v.at[pl.ds(0, 16)], [jv])[0]
                pos = plsc.load_gather(mpos_v.at[pl.ds(0, 16)], [jv])[0]
                rel = val - r0
                slot = lax.rem(issued, _RING)

                @pl.when(issued >= _RING)
                def _():
                    pltpu.make_async_copy(
                        out_hbm.at[0], oring_v.at[0], sem_out
                    ).wait()

                relv = jnp.full((16,), 0, jnp.int32) + rel
                for k in range(EMBED_DIM // 16):
                    cvec = lax.iota(jnp.int32, 16) + 16 * k
                    oring_v[slot, pl.ds(16 * k, 16)] = plsc.load_gather(
                        buf_v.at[pl.ds(0, EMBED_DIM), pl.ds(0, _W)], [cvec, relv]
                    )
                pltpu.async_copy(oring_v.at[slot], out_hbm.at[pos], sem_out)
                return issued + 1

            return lax.fori_loop(0, cnt, emit, issued)

        return lax.fori_loop(0, ng, scan_list, issued)

    issued = lax.fori_loop(0, nblk, do_block, jnp.int32(0))

    def drain(i, _):
        pltpu.make_async_copy(out_hbm.at[0], oring_v.at[0], sem_out).wait()
        return 0

    lax.fori_loop(0, jnp.minimum(issued, _RING), drain, 0)


def kernel(x, table):
    out = _gather(x.reshape(BATCH), table.T)
    return out.reshape(BATCH, 1, EMBED_DIM)


# speedup vs baseline: 2.2384x; 1.4468x over previous
import functools
import jax
import jax.numpy as jnp
from jax import lax
from jax.experimental import pallas as pl
from jax.experimental.pallas import tpu as pltpu
from jax.experimental.pallas import tpu_sc as plsc

VOCAB = 1000000
EMBED_DIM = 64
BATCH = 16384

_info = plsc.get_sparse_core_info()
_NC, _NS = _info.num_cores, _info.num_subcores
_NW = _NC * _NS
_W = 512
_NBLK = VOCAB // _W
_TAIL0 = _NBLK * _W
_TAIL = VOCAB - _TAIL0
_BASE_BLKS = _NBLK // _NW
_EXTRA = _NBLK - _BASE_BLKS * _NW
_RING = 16

_mesh = plsc.VectorSubcoreMesh(core_axis_name="c", subcore_axis_name="s")


@functools.partial(
    pl.kernel,
    mesh=_mesh,
    out_type=jax.ShapeDtypeStruct((BATCH, EMBED_DIM), jnp.float32),
    scratch_types=[
        pltpu.VMEM((BATCH,), jnp.int32),
        pltpu.VMEM((BATCH + 16,), jnp.int32),
        pltpu.VMEM((BATCH + 16,), jnp.int32),
        pltpu.VMEM((2, EMBED_DIM, _W), jnp.float32),
        pltpu.VMEM((16,), jnp.int32),
        pltpu.VMEM((16,), jnp.int32),
        pltpu.VMEM((_RING, EMBED_DIM), jnp.float32),
        pltpu.VMEM((EMBED_DIM, _TAIL), jnp.float32),
        pltpu.SemaphoreType.DMA,
        pltpu.SemaphoreType.DMA,
    ],
    compiler_params=pltpu.CompilerParams(needs_layout_passes=False),
)
def _gather(idx_hbm, tq_hbm, out_hbm, idx_all_v, lval_v, lpos_v, buf_v,
            mval_v, mpos_v, oring_v, tail_v, sem_in, sem_out):
    wid = lax.axis_index("s") * _NC + lax.axis_index("c")
    blk0 = jnp.where(
        wid < _EXTRA,
        (_BASE_BLKS + 1) * wid,
        _EXTRA * (_BASE_BLKS + 1) + _BASE_BLKS * (wid - _EXTRA),
    )
    nblk = jnp.where(wid < _EXTRA, _BASE_BLKS + 1, _BASE_BLKS)
    lo = blk0 * _W
    hi = jnp.where(wid == _NW - 1, VOCAB, (blk0 + nblk) * _W)

    pltpu.sync_copy(idx_hbm, idx_all_v)

    def scan_all(g, n):
        v = idx_all_v[pl.ds(g * 16, 16)]
        p = lax.iota(jnp.int32, 16) + g * 16
        m = jnp.logical_and(v >= lo, v < hi)
        plsc.store_compressed(lval_v.at[pl.ds(n, 16)], v, mask=m)
        plsc.store_compressed(lpos_v.at[pl.ds(n, 16)], p, mask=m)
        return n + plsc.all_reduce_population_count(m)[0]

    n = lax.fori_loop(0, BATCH // 16, scan_all, jnp.int32(0))
    ng = (n + 15) // 16

    def scan_and_emit(r0, gather_row, issued):
        def scan_list(g, issued):
            v = lval_v[pl.ds(g * 16, 16)]
            p = lpos_v[pl.ds(g * 16, 16)]
            m = jnp.logical_and(v >= r0, v < r0 + _W)
            plsc.store_compressed(mval_v.at[pl.ds(0, 16)], v, mask=m)
            plsc.store_compressed(mpos_v.at[pl.ds(0, 16)], p, mask=m)
            cnt = plsc.all_reduce_population_count(m)[0]

            def emit(j, issued):
                jv = jnp.full((16,), 0, jnp.int32) + j
                val = plsc.load_gather(mval_v.at[pl.ds(0, 16)], [jv])[0]
                pos = plsc.load_gather(mpos_v.at[pl.ds(0, 16)], [jv])[0]
                relv = jnp.full((16,), 0, jnp.int32) + (val - r0)
                oslot = lax.rem(issued, _RING)

                @pl.when(issued >= _RING)
                def _():
                    pltpu.make_async_copy(
                        out_hbm.at[0], oring_v.at[0], sem_out
                    ).wait()

                for k in range(EMBED_DIM // 16):
                    cvec = lax.iota(jnp.int32, 16) + 16 * k
                    oring_v[oslot, pl.ds(16 * k, 16)] = gather_row(cvec, relv)
                pltpu.async_copy(oring_v.at[oslot], out_hbm.at[pos], sem_out)
                return issued + 1

            return lax.fori_loop(0, cnt, emit, issued)

        return lax.fori_loop(0, ng, scan_list, issued)

    def issue_block(blk, slot):
        r0 = blk * _W
        for tc in range(EMBED_DIM // 8):
            pltpu.async_copy(
                tq_hbm.at[pl.ds(tc * 8, 8), pl.ds(r0, _W)],
                buf_v.at[slot, pl.ds(tc * 8, 8)],
                sem_in,
            )

    def drain_block(slot):
        pltpu.make_async_copy(
            tq_hbm.at[pl.ds(0, EMBED_DIM), pl.ds(0, _W)],
            buf_v.at[slot],
            sem_in,
        ).wait()

    issue_block(blk0, 0)

    def do_block(bi, issued):
        blk = blk0 + bi
        slot = lax.rem(bi, 2)
        drain_block(slot)

        @pl.when(bi + 1 < nblk)
        def _():
            issue_block(blk + 1, 1 - slot)

        slotv = jnp.full((16,), 0, jnp.int32) + slot

        def gather_row(cvec, relv):
            return plsc.load_gather(
                buf_v.at[pl.ds(0, 2), pl.ds(0, EMBED_DIM), pl.ds(0, _W)],
                [slotv, cvec, relv],
            )

        return scan_and_emit(blk * _W, gather_row, issued)

    issued = lax.fori_loop(0, nblk, do_block, jnp.int32(0))

    for tc in range(EMBED_DIM // 8):
        pltpu.sync_copy(
            tq_hbm.at[pl.ds(tc * 8, 8), pl.ds(_TAIL0, _TAIL)],
            tail_v.at[pl.ds(tc * 8, 8)],
        )

    def gather_tail(cvec, relv):
        return plsc.load_gather(
            tail_v.at[pl.ds(0, EMBED_DIM), pl.ds(0, _TAIL)], [cvec, relv]
        )

    issued = scan_and_emit(jnp.int32(_TAIL0), gather_tail, issued)

    def drain(i, _):
        pltpu.make_async_copy(out_hbm.at[0], oring_v.at[0], sem_out).wait()
        return 0

    lax.fori_loop(0, jnp.minimum(issued, _RING), drain, 0)


def kernel(x, table):
    out = _gather(x.reshape(BATCH), table.T)
    return out.reshape(BATCH, 1, EMBED_DIM)


# speedup vs baseline: 2.8047x; 1.2530x over previous
import functools
import jax
import jax.numpy as jnp
from jax import lax
from jax.experimental import pallas as pl
from jax.experimental.pallas import tpu as pltpu
from jax.experimental.pallas import tpu_sc as plsc

VOCAB = 1000000
EMBED_DIM = 64
BATCH = 16384

_info = plsc.get_sparse_core_info()
_NC, _NS = _info.num_cores, _info.num_subcores
_NW = _NC * _NS
_W = 512
_NBLK = VOCAB // _W
_TAIL0 = _NBLK * _W
_TAIL = VOCAB - _TAIL0
_BASE_BLKS = _NBLK // _NW
_EXTRA = _NBLK - _BASE_BLKS * _NW
_RING = 16

_mesh = plsc.VectorSubcoreMesh(core_axis_name="c", subcore_axis_name="s")


@functools.partial(
    pl.kernel,
    mesh=_mesh,
    out_type=jax.ShapeDtypeStruct((BATCH, EMBED_DIM), jnp.float32),
    scratch_types=[
        pltpu.VMEM((BATCH,), jnp.int32),
        pltpu.VMEM((BATCH + 16,), jnp.int32),
        pltpu.VMEM((BATCH + 16,), jnp.int32),
        pltpu.VMEM((2, EMBED_DIM, _W), jnp.float32),
        pltpu.VMEM((16,), jnp.int32),
        pltpu.VMEM((16,), jnp.int32),
        pltpu.VMEM((_RING, EMBED_DIM), jnp.float32),
        pltpu.VMEM((EMBED_DIM, _TAIL), jnp.float32),
        pltpu.SemaphoreType.DMA,
        pltpu.SemaphoreType.DMA,
    ],
    compiler_params=pltpu.CompilerParams(needs_layout_passes=False),
)
def _gather(idx_hbm, tq_hbm, out_hbm, idx_all_v, lval_v, lpos_v, buf_v,
            mval_v, mpos_v, oring_v, tail_v, sem_in, sem_out):
    wid = lax.axis_index("s") * _NC + lax.axis_index("c")
    blk0 = jnp.where(
        wid < _EXTRA,
        (_BASE_BLKS + 1) * wid,
        _EXTRA * (_BASE_BLKS + 1) + _BASE_BLKS * (wid - _EXTRA),
    )
    nblk = jnp.where(wid < _EXTRA, _BASE_BLKS + 1, _BASE_BLKS)
    lo = blk0 * _W
    hi = jnp.where(wid == _NW - 1, VOCAB, (blk0 + nblk) * _W)

    def issue_block(blk, slot):
        r0 = blk * _W
        pltpu.async_copy(
            tq_hbm.at[pl.ds(0, EMBED_DIM), pl.ds(r0, _W)],
            buf_v.at[slot],
            sem_in,
        )

    def drain_block(slot):
        pltpu.make_async_copy(
            tq_hbm.at[pl.ds(0, EMBED_DIM), pl.ds(0, _W)],
            buf_v.at[slot],
            sem_in,
        ).wait()

    issue_block(blk0, 0)
    issue_block(blk0 + 1, 1)

    pltpu.sync_copy(idx_hbm, idx_all_v)

    def scan_all(g, n):
        v = idx_all_v[pl.ds(g * 16, 16)]
        p = lax.iota(jnp.int32, 16) + g * 16
        m = jnp.logical_and(v >= lo, v < hi)
        plsc.store_compressed(lval_v.at[pl.ds(n, 16)], v, mask=m)
        plsc.store_compressed(lpos_v.at[pl.ds(n, 16)], p, mask=m)
        return n + plsc.all_reduce_population_count(m)[0]

    n = lax.fori_loop(0, BATCH // 16, scan_all, jnp.int32(0))
    ng = (n + 15) // 16

    def scan_and_emit(r0, gather_row, issued):
        def scan_list(g, issued):
            v = lval_v[pl.ds(g * 16, 16)]
            p = lpos_v[pl.ds(g * 16, 16)]
            m = jnp.logical_and(v >= r0, v < r0 + _W)
            plsc.store_compressed(mval_v.at[pl.ds(0, 16)], v, mask=m)
            plsc.store_compressed(mpos_v.at[pl.ds(0, 16)], p, mask=m)
            cnt = plsc.all_reduce_population_count(m)[0]

            def emit(j, issued):
                jv = jnp.full((16,), 0, jnp.int32) + j
                val = plsc.load_gather(mval_v.at[pl.ds(0, 16)], [jv])[0]
                pos = plsc.load_gather(mpos_v.at[pl.ds(0, 16)], [jv])[0]
                relv = jnp.full((16,), 0, jnp.int32) + (val - r0)
                oslot = lax.rem(issued, _RING)

                @pl.when(issued >= _RING)
                def _():
                    pltpu.make_async_copy(
                        out_hbm.at[0], oring_v.at[0], sem_out
                    ).wait()

                for k in range(EMBED_DIM // 16):
                    cvec = lax.iota(jnp.int32, 16) + 16 * k
                    oring_v[oslot, pl.ds(16 * k, 16)] = gather_row(cvec, relv)
                pltpu.async_copy(oring_v.at[oslot], out_hbm.at[pos], sem_out)
                return issued + 1

            return lax.fori_loop(0, cnt, emit, issued)

        return lax.fori_loop(0, ng, scan_list, issued)

    def do_block(bi, issued):
        blk = blk0 + bi
        slot = lax.rem(bi, 2)
        drain_block(slot)

        @pl.when(bi + 2 < nblk)
        def _():
            issue_block(blk + 2, slot)

        slotv = jnp.full((16,), 0, jnp.int32) + slot

        def gather_row(cvec, relv):
            return plsc.load_gather(
                buf_v.at[pl.ds(0, 2), pl.ds(0, EMBED_DIM), pl.ds(0, _W)],
                [slotv, cvec, relv],
            )

        return scan_and_emit(blk * _W, gather_row, issued)

    issued = lax.fori_loop(0, nblk, do_block, jnp.int32(0))

    for tc in range(EMBED_DIM // 8):
        pltpu.sync_copy(
            tq_hbm.at[pl.ds(tc * 8, 8), pl.ds(_TAIL0, _TAIL)],
            tail_v.at[pl.ds(tc * 8, 8)],
        )

    def gather_tail(cvec, relv):
        return plsc.load_gather(
            tail_v.at[pl.ds(0, EMBED_DIM), pl.ds(0, _TAIL)], [cvec, relv]
        )

    issued = scan_and_emit(jnp.int32(_TAIL0), gather_tail, issued)

    def drain(i, _):
        pltpu.make_async_copy(out_hbm.at[0], oring_v.at[0], sem_out).wait()
        return 0

    lax.fori_loop(0, jnp.minimum(issued, _RING), drain, 0)


def kernel(x, table):
    out = _gather(x.reshape(BATCH), table.T)
    return out.reshape(BATCH, 1, EMBED_DIM)
